# baseline (speedup 1.0000x reference)
import functools

import jax
import jax.numpy as jnp
from jax import lax
from jax.experimental import pallas as pl
from jax.experimental.pallas import tpu as pltpu
from jax.experimental.pallas import tpu_sc as plsc

NC, NS, LANES = 2, 16, 16
NW = NC * NS
EMB = 64
SEQ = 50
K = 4 * EMB
V3 = 1000
CPW = 32

_mesh = plsc.VectorSubcoreMesh(core_axis_name="c", subcore_axis_name="s",
                               num_cores=NC, num_subcores=NS)
_params = pltpu.CompilerParams(needs_layout_passes=False,
                               use_tc_tiling_on_sc=False)


def _splat(ref, j):
    return plsc.load_gather(ref, [jnp.broadcast_to(j, (LANES,)).astype(jnp.int32)])


VPAD = 1024


@functools.partial(
    pl.kernel,
    out_type=(
        jax.ShapeDtypeStruct((VPAD,), jnp.float32),
        jax.ShapeDtypeStruct((VPAD,), jnp.float32),
    ),
    mesh=_mesh,
    compiler_params=_params,
    scratch_types=[
        pltpu.VMEM((SEQ,), jnp.int32),
        pltpu.VMEM((SEQ, EMB), jnp.float32),
        pltpu.VMEM((4, LANES), jnp.float32),
        pltpu.VMEM((K,), jnp.float32),
        pltpu.VMEM((K, 128), jnp.float32),
        pltpu.VMEM((CPW,), jnp.float32),
        pltpu.VMEM((CPW,), jnp.float32),
        pltpu.VMEM((CPW,), jnp.float32),
        pltpu.SemaphoreType.DMA,
    ],
)
def _fwd_kernel(d_i, p_i, s_i, m_i, e0, e1, e2, e3, fuse_h, w_h, b_h,
                res_h, pout_h,
                idx_v, rows_v, fuse_v, rep_v, w_v, b_v, res_v, p_v, sem):
    wid = lax.axis_index("s") * NC + lax.axis_index("c")
    c0 = wid * CPW
    q0 = (wid % 4) * CPW

    pltpu.sync_copy(fuse_h, fuse_v)
    for t, (ih, eh) in enumerate(((d_i, e0), (p_i, e1), (s_i, e2), (m_i, e3))):
        pltpu.sync_copy(ih, idx_v)
        pltpu.async_copy(eh.at[idx_v], rows_v, sem).wait()

        def srow(i, accs):
            return tuple(a + rows_v[i, pl.ds(16 * c, LANES)]
                         for c, a in enumerate(accs))

        z = jnp.zeros((LANES,), jnp.float32)
        accs = lax.fori_loop(0, SEQ, srow, (z, z, z, z))
        ft = fuse_v[t]
        for c in range(4):
            rep_v[pl.ds(t * EMB + 16 * c, LANES)] = accs[c] * ft

    pltpu.sync_copy(w_h.at[:, pl.ds((wid // 4) * 128, 128)], w_v)
    pltpu.sync_copy(b_h.at[pl.ds(c0, CPW)], b_v)

    def mv(j, accs):
        a0, a1 = accs
        s = _splat(rep_v, j)
        return (a0 + s * w_v[j, pl.ds(q0, LANES)],
                a1 + s * w_v[j, pl.ds(q0 + LANES, LANES)])

    acc0, acc1 = lax.fori_loop(0, K, mv, (b_v[pl.ds(0, LANES)],
                                          b_v[pl.ds(LANES, LANES)]))
    for h, acc in ((0, acc0), (1, acc1)):
        res_v[pl.ds(16 * h, LANES)] = acc
        p_v[pl.ds(16 * h, LANES)] = 1.0 / (1.0 + jnp.exp(-acc))
    pltpu.sync_copy(res_v, res_h.at[pl.ds(c0, CPW)])
    pltpu.sync_copy(p_v, pout_h.at[pl.ds(c0, CPW)])


_NCH = 63


@functools.partial(
    pl.kernel,
    out_type=jax.ShapeDtypeStruct((NW, LANES), jnp.float32),
    mesh=_mesh,
    compiler_params=_params,
    scratch_types=[
        pltpu.VMEM((1024,), jnp.float32),
        pltpu.VMEM((CPW + 1, V3), jnp.float32),
        pltpu.VMEM((LANES,), jnp.float32),
    ],
)
def _ddi_kernel(p_h, a_h, out_h, p_v, a_v, tot_v):
    wid = lax.axis_index("s") * NC + lax.axis_index("c")
    r0 = jnp.minimum(wid * CPW, V3 - CPW)
    zero = jnp.zeros((LANES,), jnp.float32)
    p_v[pl.ds(992, LANES)] = zero
    p_v[pl.ds(1008, LANES)] = zero
    pltpu.sync_copy(p_h.at[pl.ds(0, V3)], p_v.at[pl.ds(0, V3)])
    pltpu.sync_copy(a_h.at[pl.ds(r0, CPW), :], a_v.at[pl.ds(0, CPW), :])

    def row(r, total):
        def dot(k, acc):
            return acc + a_v[r, pl.ds(16 * k, LANES)] * p_v[pl.ds(16 * k, LANES)]

        rowacc = lax.fori_loop(0, _NCH, dot, zero)
        g = r0 + r
        coef = _splat(p_v, g)
        valid = (jnp.broadcast_to(g, (LANES,)) >= wid * CPW).astype(jnp.float32)
        return total + coef * valid * rowacc

    tot_v[...] = lax.fori_loop(0, CPW, row, zero)
    pltpu.sync_copy(tot_v, out_h.at[wid])


def kernel(diag_idx, proc_idx, sym_idx, med_idx, emb0, emb1, emb2, emb3,
           fuse_w, W_q, b_q, ddi_adj):
    idx = [i.astype(jnp.int32) for i in (diag_idx, proc_idx, sym_idx, med_idx)]
    fuse16 = jnp.broadcast_to(fuse_w.reshape(4, 1), (4, LANES))
    w_pad = jnp.pad(W_q, ((0, 0), (0, VPAD - V3)))
    b_pad = jnp.pad(b_q, (0, VPAD - V3))
    res, p = _fwd_kernel(*idx, emb0, emb1, emb2, emb3, fuse16, w_pad, b_pad)
    partials = _ddi_kernel(p, ddi_adj)
    return res[None, :V3], 0.0005 * jnp.sum(partials)


# speedup vs baseline: 1.4547x; 1.4547x over previous
import functools

import jax
import jax.numpy as jnp
from jax import lax
from jax.experimental import pallas as pl
from jax.experimental.pallas import tpu as pltpu
from jax.experimental.pallas import tpu_sc as plsc

NC, NS, LANES = 2, 16, 16
NW = NC * NS
EMB = 64
SEQ = 50
K = 4 * EMB
V3 = 1000
RPW = 32

_mesh = plsc.VectorSubcoreMesh(core_axis_name="c", subcore_axis_name="s",
                               num_cores=NC, num_subcores=NS)
_sc_params = pltpu.CompilerParams(needs_layout_passes=False,
                                  use_tc_tiling_on_sc=True)


def _splat(ref, j):
    return plsc.load_gather(ref, [jnp.broadcast_to(j, (LANES,)).astype(jnp.int32)])


def _tc_body(d_i, p_i, s_i, m_i, e0, e1, e2, e3, fuse_r, w_r, b_r,
             res_o, p_o, rows_v, sem):
    idxs = (d_i, p_i, s_i, m_i)
    tabs = (e0, e1, e2, e3)
    for t in range(4):
        for j in range(SEQ):
            pltpu.make_async_copy(
                tabs[t].at[pl.ds(idxs[t][j], 1), :],
                rows_v.at[pl.ds(t * SEQ + j, 1), :],
                sem).start()
    for _ in range(4 * SEQ):
        pltpu.make_async_copy(
            tabs[0].at[pl.ds(0, 1), :], rows_v.at[pl.ds(0, 1), :], sem).wait()
    rows = rows_v[...]
    fuse = fuse_r[...]
    w = w_r[...]
    res = b_r[...]
    for t in range(4):
        rep_t = (rows[t * SEQ:(t + 1) * SEQ].sum(axis=0, keepdims=True)
                 * fuse[t:t + 1])
        res = res + jnp.dot(rep_t, w[t * EMB:(t + 1) * EMB],
                            preferred_element_type=jnp.float32)
    res_o[...] = res
    p_o[...] = 1.0 / (1.0 + jnp.exp(-res))


_tc_fwd = pl.pallas_call(
    _tc_body,
    out_shape=(jax.ShapeDtypeStruct((1, V3), jnp.float32),
               jax.ShapeDtypeStruct((1, V3), jnp.float32)),
    in_specs=[pl.BlockSpec(memory_space=pltpu.SMEM)] * 4
    + [pl.BlockSpec(memory_space=pl.ANY)] * 4
    + [pl.BlockSpec(memory_space=pltpu.VMEM)] * 3,
    out_specs=(pl.BlockSpec(memory_space=pltpu.VMEM),
               pl.BlockSpec(memory_space=pltpu.VMEM)),
    scratch_shapes=[pltpu.VMEM((4 * SEQ, EMB), jnp.float32),
                    pltpu.SemaphoreType.DMA],
)


_NFC = 62


@functools.partial(
    pl.kernel,
    out_type=jax.ShapeDtypeStruct((NW, 1, 128), jnp.float32),
    mesh=_mesh,
    compiler_params=_sc_params,
    scratch_types=[
        pltpu.VMEM((1, V3), jnp.float32),
        pltpu.VMEM((RPW, V3), jnp.float32),
        pltpu.VMEM((1, 128), jnp.float32),
    ],
)
def _ddi_kernel(p_h, a_h, out_h, p_v, a_v, tot_v):
    wid = lax.axis_index("s") * NC + lax.axis_index("c")
    r0 = jnp.minimum(wid * RPW, V3 - RPW)
    zero = jnp.zeros((LANES,), jnp.float32)
    pltpu.sync_copy(p_h, p_v)
    pltpu.sync_copy(a_h.at[pl.ds(r0, RPW), :], a_v)
    tailm = (lax.iota(jnp.int32, LANES) >= 8).astype(jnp.float32)

    def row(r, total):
        def dot(k, acc):
            return acc + a_v[r, pl.ds(16 * k, LANES)] * p_v[0, pl.ds(16 * k, LANES)]

        rowacc = lax.fori_loop(0, _NFC, dot, zero)
        rowacc = rowacc + (a_v[r, pl.ds(984, LANES)] * tailm
                           * p_v[0, pl.ds(984, LANES)])
        g = r0 + r
        coef = _splat(p_v.at[0], g)
        valid = (jnp.broadcast_to(g, (LANES,)) >= wid * RPW).astype(jnp.float32)
        return total + coef * valid * rowacc

    for m in range(1, 8):
        tot_v[0, pl.ds(16 * m, LANES)] = zero
    tot_v[0, pl.ds(0, LANES)] = lax.fori_loop(0, RPW, row, zero)
    pltpu.sync_copy(tot_v, out_h.at[wid])


def kernel(diag_idx, proc_idx, sym_idx, med_idx, emb0, emb1, emb2, emb3,
           fuse_w, W_q, b_q, ddi_adj):
    idx = [i.astype(jnp.int32) for i in (diag_idx, proc_idx, sym_idx, med_idx)]
    fuse64 = jnp.broadcast_to(fuse_w.reshape(4, 1), (4, EMB))
    res, p = _tc_fwd(*idx, emb0, emb1, emb2, emb3, fuse64, W_q, b_q[None, :])
    partials = _ddi_kernel(p, ddi_adj)
    return res, 0.0005 * jnp.sum(partials)
